# Initial kernel scaffold; baseline (speedup 1.0000x reference)
import functools

import jax
import jax.numpy as jnp
from jax import lax
from jax.experimental import pallas as pl
from jax.experimental.pallas import tpu as pltpu
from jax.experimental.pallas import tpu_sc as plsc

N = 10000
E = 320000
ND = 128
ED = 16
D = 128

NC = 2
NS = 16
L = 16
NW = NC * NS
EPW = E // NW
K = 80
NCHUNK = EPW // K
RPT = N // NS
ZR = 25


def _t1_body(x_ref, we_ref, be_ref, w1_ref, w2_ref, nf_ref, a_ref, b_ref):
    nf = jnp.dot(x_ref[...], we_ref[...], preferred_element_type=jnp.float32)
    nf = nf + be_ref[...]
    nf_ref[...] = nf
    a_ref[...] = jnp.dot(nf, w1_ref[...], preferred_element_type=jnp.float32)
    b_ref[...] = jnp.dot(nf, w2_ref[...], preferred_element_type=jnp.float32)


def _t2_body(ea_ref, we_ref, be_ref, w3_ref, b3_ref, ef_ref, cb_ref):
    ef = jnp.dot(ea_ref[...], we_ref[...], preferred_element_type=jnp.float32)
    ef = ef + be_ref[...]
    ef_ref[...] = ef
    cb_ref[...] = (
        jnp.dot(ef, w3_ref[...], preferred_element_type=jnp.float32) + b3_ref[...]
    )


def _t3_body(nf_ref, a0_ref, a1_ref, wn1_ref, wn2_ref, bn_ref, out_ref):
    nf = nf_ref[...]
    agg = a0_ref[...] + a1_ref[...]
    h = (
        jnp.dot(nf, wn1_ref[...], preferred_element_type=jnp.float32)
        + jnp.dot(agg, wn2_ref[...], preferred_element_type=jnp.float32)
        + bn_ref[...]
    )
    out_ref[...] = nf + jnp.maximum(h, 0.0)


def _rep(spec, n):
    return [spec] * n


def _t1(x, We, be, W1, W2):
    blk = 2000
    return pl.pallas_call(
        _t1_body,
        grid=(N // blk,),
        in_specs=[
            pl.BlockSpec((blk, ND), lambda i: (i, 0)),
            pl.BlockSpec((ND, D), lambda i: (0, 0)),
            pl.BlockSpec((1, D), lambda i: (0, 0)),
            pl.BlockSpec((D, D), lambda i: (0, 0)),
            pl.BlockSpec((D, D), lambda i: (0, 0)),
        ],
        out_specs=_rep(pl.BlockSpec((blk, D), lambda i: (i, 0)), 3),
        out_shape=_rep(jax.ShapeDtypeStruct((N, D), jnp.float32), 3),
    )(x, We, be, W1, W2)


def _t2(ea, We, be, W3, b3):
    blk = 8000
    return pl.pallas_call(
        _t2_body,
        grid=(E // blk,),
        in_specs=[
            pl.BlockSpec((blk, ED), lambda i: (i, 0)),
            pl.BlockSpec((ED, D), lambda i: (0, 0)),
            pl.BlockSpec((1, D), lambda i: (0, 0)),
            pl.BlockSpec((D, D), lambda i: (0, 0)),
            pl.BlockSpec((1, D), lambda i: (0, 0)),
        ],
        out_specs=_rep(pl.BlockSpec((blk, D), lambda i: (i, 0)), 2),
        out_shape=_rep(jax.ShapeDtypeStruct((E, D), jnp.float32), 2),
    )(ea, We, be, W3, b3)


def _t3(nf, a0, a1, Wn1, Wn2, bn):
    blk = 2000
    return pl.pallas_call(
        _t3_body,
        grid=(N // blk,),
        in_specs=[
            pl.BlockSpec((blk, D), lambda i: (i, 0)),
            pl.BlockSpec((blk, D), lambda i: (i, 0)),
            pl.BlockSpec((blk, D), lambda i: (i, 0)),
            pl.BlockSpec((D, D), lambda i: (0, 0)),
            pl.BlockSpec((D, D), lambda i: (0, 0)),
            pl.BlockSpec((1, D), lambda i: (0, 0)),
        ],
        out_specs=pl.BlockSpec((blk, D), lambda i: (i, 0)),
        out_shape=jax.ShapeDtypeStruct((N, D), jnp.float32),
    )(nf, a0, a1, Wn1, Wn2, bn)


def _sc_body(src_hbm, dst_hbm, a_hbm, b_hbm, ef_hbm, cb_hbm,
             efnew_hbm, agg_hbm,
             src_v, dst_v, a_v, b_v, ef_v, cb_v, zero_v, agg_sp,
             sem_a, sem_b):
    c = lax.axis_index("c")
    s = lax.axis_index("s")
    w = s * NC + c

    def zfill(i, carry):
        for j in range(D // L):
            zero_v[i, pl.ds(j * L, L)] = jnp.zeros((L,), jnp.float32)
        return carry

    lax.fori_loop(0, ZR, zfill, 0)

    base_row = s * RPT

    def zcopy(k, carry):
        pltpu.sync_copy(zero_v, agg_sp.at[pl.ds(base_row + k * ZR, ZR)])
        return carry

    lax.fori_loop(0, RPT // ZR, zcopy, 0)
    plsc.subcore_barrier()

    pltpu.sync_copy(src_hbm.at[w], src_v)
    pltpu.sync_copy(dst_hbm.at[w], dst_v)

    ebase = w * EPW

    def chunk(g, carry):
        cp_a = pltpu.async_copy(a_hbm.at[src_v.at[g]], a_v, sem_a)
        cp_b = pltpu.async_copy(b_hbm.at[dst_v.at[g]], b_v, sem_b)
        row0 = ebase + g * K
        pltpu.sync_copy(ef_hbm.at[pl.ds(row0, K)], ef_v)
        pltpu.sync_copy(cb_hbm.at[pl.ds(row0, K)], cb_v)
        cp_a.wait()
        cp_b.wait()

        def crow(r, carry2):
            for j in range(D // L):
                sl = pl.ds(j * L, L)
                acc = a_v[r, sl] + b_v[r, sl] + cb_v[r, sl]
                a_v[r, sl] = ef_v[r, sl] + jnp.maximum(
                    acc, jnp.zeros((L,), jnp.float32))
            return carry2

        lax.fori_loop(0, K, crow, 0)

        pltpu.sync_copy(a_v, efnew_hbm.at[pl.ds(row0, K)])
        pltpu.sync_copy(a_v, agg_sp.at[dst_v.at[g]], add=True)
        return carry

    lax.fori_loop(0, NCHUNK, chunk, 0)

    plsc.subcore_barrier()

    def ocopy(k, carry):
        r0 = base_row + k * ZR
        pltpu.sync_copy(agg_sp.at[pl.ds(r0, ZR)], agg_hbm.at[c, pl.ds(r0, ZR)])
        return carry

    lax.fori_loop(0, RPT // ZR, ocopy, 0)


_sc_call = functools.partial(
    pl.kernel,
    out_type=(
        jax.ShapeDtypeStruct((E, D), jnp.float32),
        jax.ShapeDtypeStruct((NC, N, D), jnp.float32),
    ),
    mesh=plsc.VectorSubcoreMesh(core_axis_name="c", subcore_axis_name="s",
                                num_cores=NC, num_subcores=NS),
    scratch_types=[
        pltpu.VMEM((NCHUNK, K), jnp.int32),
        pltpu.VMEM((NCHUNK, K), jnp.int32),
        pltpu.VMEM((K, D), jnp.float32),
        pltpu.VMEM((K, D), jnp.float32),
        pltpu.VMEM((K, D), jnp.float32),
        pltpu.VMEM((K, D), jnp.float32),
        pltpu.VMEM((ZR, D), jnp.float32),
        pltpu.VMEM_SHARED((N, D), jnp.float32),
        pltpu.SemaphoreType.DMA,
        pltpu.SemaphoreType.DMA,
    ],
)(_sc_body)


def kernel(x, edge_attr, edge_index, W_node_enc, b_node_enc, W_edge_enc,
           b_edge_enc, W_edge, b_edge, W_node, b_node):
    src = edge_index[0].reshape(NW, NCHUNK, K)
    dst = edge_index[1].reshape(NW, NCHUNK, K)
    W1 = W_edge[:D]
    W2 = W_edge[D:2 * D]
    W3 = W_edge[2 * D:]

    nf, A, B = _t1(x, W_node_enc, b_node_enc.reshape(1, D), W1, W2)
    ef, cb = _t2(edge_attr, W_edge_enc, b_edge_enc.reshape(1, D), W3,
                 b_edge.reshape(1, D))
    ef_new, agg2 = _sc_call(src, dst, A, B, ef, cb)
    nf_new = _t3(nf, agg2[0], agg2[1], W_node[:D], W_node[D:],
                 b_node.reshape(1, D))
    return nf_new, ef_new


# speedup vs baseline: 2.7548x; 2.7548x over previous
import functools

import jax
import jax.numpy as jnp
from jax import lax
from jax.experimental import pallas as pl
from jax.experimental.pallas import tpu as pltpu
from jax.experimental.pallas import tpu_sc as plsc

N = 10000
E = 320000
ND = 128
ED = 16
D = 128

NC = 2
NS = 16
L = 16
NW = NC * NS
EPW = E // NW
K = 80
NCHUNK = EPW // K
IB = 25
NB = NCHUNK // IB
AGG_ROWS = 10240
RPT = AGG_ROWS // NS
ZR = 32


def _t1_body(x_ref, we_ref, be_ref, w1_ref, w2_ref, nf_ref, a_ref, b_ref):
    nf = jnp.dot(x_ref[...], we_ref[...], preferred_element_type=jnp.float32)
    nf = nf + be_ref[...]
    nf_ref[...] = nf
    a_ref[...] = jnp.dot(nf, w1_ref[...], preferred_element_type=jnp.float32)
    b_ref[...] = jnp.dot(nf, w2_ref[...], preferred_element_type=jnp.float32)


def _t2_body(ea_ref, we_ref, be_ref, w3_ref, b3_ref, ef_ref, cb_ref):
    ef = jnp.dot(ea_ref[...], we_ref[...], preferred_element_type=jnp.float32)
    ef = ef + be_ref[...]
    ef_ref[...] = ef
    cb_ref[...] = (
        jnp.dot(ef, w3_ref[...], preferred_element_type=jnp.float32) + b3_ref[...]
    )


def _t3_body(nf_ref, a0_ref, a1_ref, wn1_ref, wn2_ref, bn_ref, out_ref):
    nf = nf_ref[...]
    agg = a0_ref[...] + a1_ref[...]
    h = (
        jnp.dot(nf, wn1_ref[...], preferred_element_type=jnp.float32)
        + jnp.dot(agg, wn2_ref[...], preferred_element_type=jnp.float32)
        + bn_ref[...]
    )
    out_ref[...] = nf + jnp.maximum(h, 0.0)


def _rep(spec, n):
    return [spec] * n


def _t1(x, We, be, W1, W2):
    blk = 2000
    return pl.pallas_call(
        _t1_body,
        grid=(N // blk,),
        in_specs=[
            pl.BlockSpec((blk, ND), lambda i: (i, 0)),
            pl.BlockSpec((ND, D), lambda i: (0, 0)),
            pl.BlockSpec((1, D), lambda i: (0, 0)),
            pl.BlockSpec((D, D), lambda i: (0, 0)),
            pl.BlockSpec((D, D), lambda i: (0, 0)),
        ],
        out_specs=_rep(pl.BlockSpec((blk, D), lambda i: (i, 0)), 3),
        out_shape=_rep(jax.ShapeDtypeStruct((N, D), jnp.float32), 3),
    )(x, We, be, W1, W2)


def _t2(ea, We, be, W3, b3):
    blk = 8000
    return pl.pallas_call(
        _t2_body,
        grid=(E // blk,),
        in_specs=[
            pl.BlockSpec((blk, ED), lambda i: (i, 0)),
            pl.BlockSpec((ED, D), lambda i: (0, 0)),
            pl.BlockSpec((1, D), lambda i: (0, 0)),
            pl.BlockSpec((D, D), lambda i: (0, 0)),
            pl.BlockSpec((1, D), lambda i: (0, 0)),
        ],
        out_specs=_rep(pl.BlockSpec((blk, D), lambda i: (i, 0)), 2),
        out_shape=_rep(jax.ShapeDtypeStruct((E, D), jnp.float32), 2),
    )(ea, We, be, W3, b3)


def _t3(nf, a0, a1, Wn1, Wn2, bn):
    blk = 2000
    return pl.pallas_call(
        _t3_body,
        grid=(N // blk,),
        in_specs=[
            pl.BlockSpec((blk, D), lambda i: (i, 0)),
            pl.BlockSpec((blk, D), lambda i: (i, 0)),
            pl.BlockSpec((blk, D), lambda i: (i, 0)),
            pl.BlockSpec((D, D), lambda i: (0, 0)),
            pl.BlockSpec((D, D), lambda i: (0, 0)),
            pl.BlockSpec((1, D), lambda i: (0, 0)),
        ],
        out_specs=pl.BlockSpec((blk, D), lambda i: (i, 0)),
        out_shape=jax.ShapeDtypeStruct((N, D), jnp.float32),
    )(nf, a0, a1, Wn1, Wn2, bn)


def _sc_body(src_hbm, dst_hbm, a_hbm, b_hbm, ef_hbm, cb_hbm,
             efnew_hbm, agg_hbm,
             src_v, dst_v, a_v, b_v, ef_v, cb_v, agg_sp,
             sem_a, sem_b):
    c = lax.axis_index("c")
    s = lax.axis_index("s")
    w = s * NC + c

    def zfill(i, carry):
        for j in range(D // L):
            ef_v[i, pl.ds(j * L, L)] = jnp.zeros((L,), jnp.float32)
        return carry

    lax.fori_loop(0, ZR, zfill, 0)

    base_row = s * RPT

    def zcopy(k, carry):
        pltpu.sync_copy(ef_v.at[pl.ds(0, ZR)],
                        agg_sp.at[pl.ds(base_row + k * ZR, ZR)])
        return carry

    lax.fori_loop(0, RPT // ZR, zcopy, 0)
    plsc.subcore_barrier()

    ebase = w * EPW

    def batch(bidx, carry):
        pltpu.sync_copy(src_hbm.at[w, bidx], src_v)
        pltpu.sync_copy(dst_hbm.at[w, bidx], dst_v)

        def chunk(j, carry2):
            g = bidx * IB + j
            cp_a = pltpu.async_copy(a_hbm.at[src_v.at[j]], a_v, sem_a)
            cp_b = pltpu.async_copy(b_hbm.at[dst_v.at[j]], b_v, sem_b)
            row0 = ebase + g * K
            pltpu.sync_copy(ef_hbm.at[pl.ds(row0, K)], ef_v)
            pltpu.sync_copy(cb_hbm.at[pl.ds(row0, K)], cb_v)
            cp_a.wait()
            cp_b.wait()

            def crow(r, carry3):
                for jj in range(D // L):
                    sl = pl.ds(jj * L, L)
                    acc = a_v[r, sl] + b_v[r, sl] + cb_v[r, sl]
                    a_v[r, sl] = ef_v[r, sl] + jnp.maximum(
                        acc, jnp.zeros((L,), jnp.float32))
                return carry3

            lax.fori_loop(0, K, crow, 0)

            pltpu.sync_copy(a_v, efnew_hbm.at[pl.ds(row0, K)])
            pltpu.sync_copy(a_v, agg_sp.at[dst_v.at[j]], add=True)
            return carry2

        lax.fori_loop(0, IB, chunk, 0)
        return carry

    lax.fori_loop(0, NB, batch, 0)

    plsc.subcore_barrier()

    def ocopy(k, carry):
        r0 = base_row + k * ZR
        pltpu.sync_copy(agg_sp.at[pl.ds(r0, ZR)], agg_hbm.at[c, pl.ds(r0, ZR)])
        return carry

    lax.fori_loop(0, RPT // ZR, ocopy, 0)


@functools.cache
def _sc_call():
    return functools.partial(
        pl.kernel,
        out_type=(
            jax.ShapeDtypeStruct((E, D), jnp.float32),
            jax.ShapeDtypeStruct((NC, AGG_ROWS, D), jnp.float32),
        ),
        mesh=plsc.VectorSubcoreMesh(core_axis_name="c", subcore_axis_name="s",
                                    num_cores=NC, num_subcores=NS),
        scratch_types=[
            pltpu.VMEM((IB, K), jnp.int32),
            pltpu.VMEM((IB, K), jnp.int32),
            pltpu.VMEM((K, D), jnp.float32),
            pltpu.VMEM((K, D), jnp.float32),
            pltpu.VMEM((K, D), jnp.float32),
            pltpu.VMEM((K, D), jnp.float32),
            pltpu.VMEM_SHARED((AGG_ROWS, D), jnp.float32),
            pltpu.SemaphoreType.DMA,
            pltpu.SemaphoreType.DMA,
        ],
    )(_sc_body)


def kernel(x, edge_attr, edge_index, W_node_enc, b_node_enc, W_edge_enc,
           b_edge_enc, W_edge, b_edge, W_node, b_node):
    src = edge_index[0].reshape(NW, NB, IB, K)
    dst = edge_index[1].reshape(NW, NB, IB, K)
    W1 = W_edge[:D]
    W2 = W_edge[D:2 * D]
    W3 = W_edge[2 * D:]

    nf, A, B = _t1(x, W_node_enc, b_node_enc.reshape(1, D), W1, W2)
    ef, cb = _t2(edge_attr, W_edge_enc, b_edge_enc.reshape(1, D), W3,
                 b_edge.reshape(1, D))
    ef_new, agg2 = _sc_call()(src, dst, A, B, ef, cb)
    nf_new = _t3(nf, agg2[0, :N], agg2[1, :N], W_node[:D], W_node[D:],
                 b_node.reshape(1, D))
    return nf_new, ef_new


# speedup vs baseline: 3.2854x; 1.1926x over previous
import functools

import jax
import jax.numpy as jnp
from jax import lax
from jax.experimental import pallas as pl
from jax.experimental.pallas import tpu as pltpu
from jax.experimental.pallas import tpu_sc as plsc

N = 10000
E = 320000
ND = 128
ED = 16
D = 128
DH = D // 2

NC = 2
NS = 16
L = 16
NW = NC * NS
EPW = E // NW
K = 80
NCHUNK = EPW // K
IB = 25
NB = NCHUNK // IB
AGG_ROWS = 10240
RPT = AGG_ROWS // NS
ZR = 32

_COLS_E = [32 * j + m for j in range(4) for m in range(16)]
_COLS_O = [c + 16 for c in _COLS_E]


def _pack_pair(ve, vo):
    he = lax.bitcast_convert_type(ve.astype(jnp.bfloat16), jnp.uint16)
    ho = lax.bitcast_convert_type(vo.astype(jnp.bfloat16), jnp.uint16)
    packed = he.astype(jnp.uint32) | (ho.astype(jnp.uint32) << 16)
    return lax.bitcast_convert_type(packed, jnp.int32)


def _t1_body(x_ref, we_ref, be_ref, w1_ref, w2_ref, nf_ref, a_ref, b_ref):
    nf = jnp.dot(x_ref[...], we_ref[...], preferred_element_type=jnp.float32)
    nf = nf + be_ref[...]
    nf_ref[...] = nf
    a_ref[...] = jnp.dot(nf, w1_ref[...], preferred_element_type=jnp.float32)
    b_ref[...] = jnp.dot(nf, w2_ref[...], preferred_element_type=jnp.float32)


def _t2_body(ea_ref, we_ref, be_ref, wee_ref, weo_ref, bee_ref, beo_ref,
             w3e_ref, w3o_ref, b3e_ref, b3o_ref, efcb_ref):
    ea = ea_ref[...]
    ef = jnp.dot(ea, we_ref[...], preferred_element_type=jnp.float32)
    ef = ef + be_ref[...]
    ef_e = jnp.dot(ea, wee_ref[...], preferred_element_type=jnp.float32)
    ef_o = jnp.dot(ea, weo_ref[...], preferred_element_type=jnp.float32)
    efcb_ref[:, :DH] = _pack_pair(ef_e + bee_ref[...], ef_o + beo_ref[...])
    cb_e = jnp.dot(ef, w3e_ref[...], preferred_element_type=jnp.float32)
    cb_o = jnp.dot(ef, w3o_ref[...], preferred_element_type=jnp.float32)
    efcb_ref[:, DH:] = _pack_pair(cb_e + b3e_ref[...], cb_o + b3o_ref[...])


def _t3_body(nf_ref, a0_ref, a1_ref, wn1_ref, wn2_ref, bn_ref, out_ref):
    nf = nf_ref[...]
    agg = a0_ref[...] + a1_ref[...]
    h = (
        jnp.dot(nf, wn1_ref[...], preferred_element_type=jnp.float32)
        + jnp.dot(agg, wn2_ref[...], preferred_element_type=jnp.float32)
        + bn_ref[...]
    )
    out_ref[...] = nf + jnp.maximum(h, 0.0)


def _full(shape):
    return pl.BlockSpec(shape, lambda i: (0, 0))


def _t1(x, We, be, W1, W2):
    blk = 2000
    row = lambda d: pl.BlockSpec((blk, d), lambda i: (i, 0))
    return pl.pallas_call(
        _t1_body,
        grid=(N // blk,),
        in_specs=[row(ND), _full((ND, D)), _full((1, D)),
                  _full((D, D)), _full((D, D))],
        out_specs=[row(D), row(D), row(D)],
        out_shape=[jax.ShapeDtypeStruct((N, D), jnp.float32),
                   jax.ShapeDtypeStruct((N, D), jnp.float32),
                   jax.ShapeDtypeStruct((N, D), jnp.float32)],
    )(x, We, be, W1, W2)


def _t2(ea, We, be, Wee, Weo, bee, beo, W3e, W3o, b3e, b3o):
    blk = 8000
    return pl.pallas_call(
        _t2_body,
        grid=(E // blk,),
        in_specs=[pl.BlockSpec((blk, ED), lambda i: (i, 0)),
                  _full((ED, D)), _full((1, D)),
                  _full((ED, DH)), _full((ED, DH)),
                  _full((1, DH)), _full((1, DH)),
                  _full((D, DH)), _full((D, DH)),
                  _full((1, DH)), _full((1, DH))],
        out_specs=pl.BlockSpec((blk, D), lambda i: (i, 0)),
        out_shape=jax.ShapeDtypeStruct((E, D), jnp.int32),
    )(ea, We, be, Wee, Weo, bee, beo, W3e, W3o, b3e, b3o)


def _t3(nf, a0, a1, Wn1, Wn2, bn):
    blk = 2000
    row = lambda d: pl.BlockSpec((blk, d), lambda i: (i, 0))
    return pl.pallas_call(
        _t3_body,
        grid=(N // blk,),
        in_specs=[row(D), row(D), row(D),
                  _full((D, D)), _full((D, D)), _full((1, D))],
        out_specs=row(D),
        out_shape=jax.ShapeDtypeStruct((N, D), jnp.float32),
    )(nf, a0, a1, Wn1, Wn2, bn)


def _sc_body(src_hbm, dst_hbm, a_hbm, b_hbm, efcb_hbm,
             efnew_hbm, agg_hbm,
             src_v, dst_v, a_v, b_v, efcb_v, out_v, agg_sp,
             sem_a, sem_b):
    c = lax.axis_index("c")
    s = lax.axis_index("s")
    w = s * NC + c

    def zfill(i, carry):
        for j in range(D // L):
            out_v[i, pl.ds(j * L, L)] = jnp.zeros((L,), jnp.float32)
        return carry

    lax.fori_loop(0, ZR, zfill, 0)

    base_row = s * RPT

    def zcopy(k, carry):
        pltpu.sync_copy(out_v.at[pl.ds(0, ZR)],
                        agg_sp.at[pl.ds(base_row + k * ZR, ZR)])
        return carry

    lax.fori_loop(0, RPT // ZR, zcopy, 0)
    plsc.subcore_barrier()

    ebase = w * EPW

    def unpack2(ref, r, grp):
        v = ref[r, pl.ds(grp * L, L)]
        lo = lax.bitcast_convert_type(lax.shift_left(v, 16), jnp.float32)
        hi = lax.bitcast_convert_type(v & jnp.int32(-65536), jnp.float32)
        return lo, hi

    def batch(bidx, carry):
        pltpu.sync_copy(src_hbm.at[w, bidx], src_v)
        pltpu.sync_copy(dst_hbm.at[w, bidx], dst_v)

        def chunk(j, carry2):
            g = bidx * IB + j
            cp_a = pltpu.async_copy(a_hbm.at[src_v.at[j]], a_v, sem_a)
            cp_b = pltpu.async_copy(b_hbm.at[dst_v.at[j]], b_v, sem_b)
            row0 = ebase + g * K
            pltpu.sync_copy(efcb_hbm.at[pl.ds(row0, K)], efcb_v)
            cp_a.wait()
            cp_b.wait()

            def crow(r, carry3):
                for grp in range(4):
                    a_lo = a_v[r, pl.ds(32 * grp, L)]
                    a_hi = a_v[r, pl.ds(32 * grp + L, L)]
                    b_lo = b_v[r, pl.ds(32 * grp, L)]
                    b_hi = b_v[r, pl.ds(32 * grp + L, L)]
                    ef_lo, ef_hi = unpack2(efcb_v, r, grp)
                    cb_lo, cb_hi = unpack2(efcb_v, r, grp + 4)
                    zero = jnp.zeros((L,), jnp.float32)
                    out_v[r, pl.ds(32 * grp, L)] = ef_lo + jnp.maximum(
                        a_lo + b_lo + cb_lo, zero)
                    out_v[r, pl.ds(32 * grp + L, L)] = ef_hi + jnp.maximum(
                        a_hi + b_hi + cb_hi, zero)
                return carry3

            lax.fori_loop(0, K, crow, 0)

            pltpu.sync_copy(out_v, efnew_hbm.at[pl.ds(row0, K)])
            pltpu.sync_copy(out_v, agg_sp.at[dst_v.at[j]], add=True)
            return carry2

        lax.fori_loop(0, IB, chunk, 0)
        return carry

    lax.fori_loop(0, NB, batch, 0)

    plsc.subcore_barrier()

    def ocopy(k, carry):
        r0 = base_row + k * ZR
        pltpu.sync_copy(agg_sp.at[pl.ds(r0, ZR)], agg_hbm.at[c, pl.ds(r0, ZR)])
        return carry

    lax.fori_loop(0, RPT // ZR, ocopy, 0)


@functools.cache
def _sc_call():
    return functools.partial(
        pl.kernel,
        out_type=(
            jax.ShapeDtypeStruct((E, D), jnp.float32),
            jax.ShapeDtypeStruct((NC, AGG_ROWS, D), jnp.float32),
        ),
        mesh=plsc.VectorSubcoreMesh(core_axis_name="c", subcore_axis_name="s",
                                    num_cores=NC, num_subcores=NS),
        scratch_types=[
            pltpu.VMEM((IB, K), jnp.int32),
            pltpu.VMEM((IB, K), jnp.int32),
            pltpu.VMEM((K, D), jnp.float32),
            pltpu.VMEM((K, D), jnp.float32),
            pltpu.VMEM((K, D), jnp.int32),
            pltpu.VMEM((K, D), jnp.float32),
            pltpu.VMEM_SHARED((AGG_ROWS, D), jnp.float32),
            pltpu.SemaphoreType.DMA,
            pltpu.SemaphoreType.DMA,
        ],
    )(_sc_body)


def kernel(x, edge_attr, edge_index, W_node_enc, b_node_enc, W_edge_enc,
           b_edge_enc, W_edge, b_edge, W_node, b_node):
    src = edge_index[0].reshape(NW, NB, IB, K)
    dst = edge_index[1].reshape(NW, NB, IB, K)
    ce = jnp.array(_COLS_E, dtype=jnp.int32)
    co = jnp.array(_COLS_O, dtype=jnp.int32)
    W1 = W_edge[:D]
    W2 = W_edge[D:2 * D]
    W3 = W_edge[2 * D:]

    nf, A, B = _t1(x, W_node_enc, b_node_enc.reshape(1, D), W1, W2)
    efcb = _t2(edge_attr, W_edge_enc, b_edge_enc.reshape(1, D),
               W_edge_enc[:, ce], W_edge_enc[:, co],
               b_edge_enc[ce].reshape(1, DH), b_edge_enc[co].reshape(1, DH),
               W3[:, ce], W3[:, co],
               b_edge[ce].reshape(1, DH), b_edge[co].reshape(1, DH))
    ef_new, agg2 = _sc_call()(src, dst, A, B, efcb)
    nf_new = _t3(nf, agg2[0, :N], agg2[1, :N], W_node[:D], W_node[D:],
                 b_node.reshape(1, D))
    return nf_new, ef_new


# speedup vs baseline: 3.3038x; 1.0056x over previous
import functools

import jax
import jax.numpy as jnp
from jax import lax
from jax.experimental import pallas as pl
from jax.experimental.pallas import tpu as pltpu
from jax.experimental.pallas import tpu_sc as plsc

N = 10000
E = 320000
ND = 128
ED = 16
D = 128
DH = D // 2

NC = 2
NS = 16
L = 16
NW = NC * NS
EPW = E // NW
K = 40
NCHUNK = EPW // K
IB = 50
NB = NCHUNK // IB
HPB = IB // 2
AGG_ROWS = 10112
RPT = AGG_ROWS // NS
ZR = 8

_COLS_E = [32 * j + m for j in range(4) for m in range(16)]
_COLS_O = [c + 16 for c in _COLS_E]


def _pack_pair(ve, vo):
    he = lax.bitcast_convert_type(ve.astype(jnp.bfloat16), jnp.uint16)
    ho = lax.bitcast_convert_type(vo.astype(jnp.bfloat16), jnp.uint16)
    packed = he.astype(jnp.uint32) | (ho.astype(jnp.uint32) << 16)
    return lax.bitcast_convert_type(packed, jnp.int32)


def _t1_body(x_ref, we_ref, be_ref, w1_ref, w2_ref, nf_ref, a_ref, b_ref):
    nf = jnp.dot(x_ref[...], we_ref[...], preferred_element_type=jnp.float32)
    nf = nf + be_ref[...]
    nf_ref[...] = nf
    a_ref[...] = jnp.dot(nf, w1_ref[...], preferred_element_type=jnp.float32)
    b_ref[...] = jnp.dot(nf, w2_ref[...], preferred_element_type=jnp.float32)


def _t2_body(ea_ref, we_ref, be_ref, wee_ref, weo_ref, bee_ref, beo_ref,
             w3e_ref, w3o_ref, b3e_ref, b3o_ref, efcb_ref):
    ea = ea_ref[...]
    ef = jnp.dot(ea, we_ref[...], preferred_element_type=jnp.float32)
    ef = ef + be_ref[...]
    ef_e = jnp.dot(ea, wee_ref[...], preferred_element_type=jnp.float32)
    ef_o = jnp.dot(ea, weo_ref[...], preferred_element_type=jnp.float32)
    efcb_ref[:, :DH] = _pack_pair(ef_e + bee_ref[...], ef_o + beo_ref[...])
    cb_e = jnp.dot(ef, w3e_ref[...], preferred_element_type=jnp.float32)
    cb_o = jnp.dot(ef, w3o_ref[...], preferred_element_type=jnp.float32)
    efcb_ref[:, DH:] = _pack_pair(cb_e + b3e_ref[...], cb_o + b3o_ref[...])


def _t3_body(nf_ref, a0_ref, a1_ref, wn1_ref, wn2_ref, bn_ref, out_ref):
    nf = nf_ref[...]
    agg = a0_ref[...] + a1_ref[...]
    h = (
        jnp.dot(nf, wn1_ref[...], preferred_element_type=jnp.float32)
        + jnp.dot(agg, wn2_ref[...], preferred_element_type=jnp.float32)
        + bn_ref[...]
    )
    out_ref[...] = nf + jnp.maximum(h, 0.0)


def _full(shape):
    return pl.BlockSpec(shape, lambda i: (0, 0))


def _t1(x, We, be, W1, W2):
    blk = 2000
    row = lambda d: pl.BlockSpec((blk, d), lambda i: (i, 0))
    return pl.pallas_call(
        _t1_body,
        grid=(N // blk,),
        in_specs=[row(ND), _full((ND, D)), _full((1, D)),
                  _full((D, D)), _full((D, D))],
        out_specs=[row(D), row(D), row(D)],
        out_shape=[jax.ShapeDtypeStruct((N, D), jnp.float32),
                   jax.ShapeDtypeStruct((N, D), jnp.float32),
                   jax.ShapeDtypeStruct((N, D), jnp.float32)],
    )(x, We, be, W1, W2)


def _t2(ea, We, be, Wee, Weo, bee, beo, W3e, W3o, b3e, b3o):
    blk = 8000
    return pl.pallas_call(
        _t2_body,
        grid=(E // blk,),
        in_specs=[pl.BlockSpec((blk, ED), lambda i: (i, 0)),
                  _full((ED, D)), _full((1, D)),
                  _full((ED, DH)), _full((ED, DH)),
                  _full((1, DH)), _full((1, DH)),
                  _full((D, DH)), _full((D, DH)),
                  _full((1, DH)), _full((1, DH))],
        out_specs=pl.BlockSpec((blk, D), lambda i: (i, 0)),
        out_shape=jax.ShapeDtypeStruct((E, D), jnp.int32),
    )(ea, We, be, Wee, Weo, bee, beo, W3e, W3o, b3e, b3o)


def _t3(nf, a0, a1, Wn1, Wn2, bn):
    blk = 2000
    row = lambda d: pl.BlockSpec((blk, d), lambda i: (i, 0))
    return pl.pallas_call(
        _t3_body,
        grid=(N // blk,),
        in_specs=[row(D), row(D), row(D),
                  _full((D, D)), _full((D, D)), _full((1, D))],
        out_specs=row(D),
        out_shape=jax.ShapeDtypeStruct((N, D), jnp.float32),
    )(nf, a0, a1, Wn1, Wn2, bn)


def _sc_body(eidx_hbm, a_hbm, b_hbm, efcb_hbm,
             efnew_hbm, agg_hbm,
             idx_v, a_v, b_v, efcb_v, out_v, agg_sp,
             sem_a0, sem_a1, sem_b0, sem_b1, sem_e, sem_w0, sem_w1,
             sem_s0, sem_s1):
    c = lax.axis_index("c")
    s = lax.axis_index("s")
    w = s * NC + c
    sem_a = (sem_a0, sem_a1)
    sem_b = (sem_b0, sem_b1)
    sem_w = (sem_w0, sem_w1)
    sem_s = (sem_s0, sem_s1)

    def zfill(i, carry):
        for j in range(D // L):
            out_v[0, i, pl.ds(j * L, L)] = jnp.zeros((L,), jnp.float32)
        return carry

    lax.fori_loop(0, ZR, zfill, 0)

    base_row = s * RPT

    def zcopy(k, carry):
        pltpu.sync_copy(out_v.at[0, pl.ds(0, ZR)],
                        agg_sp.at[pl.ds(base_row + k * ZR, ZR)])
        return carry

    lax.fori_loop(0, RPT // ZR, zcopy, 0)
    plsc.subcore_barrier()

    ebase = w * EPW

    def unpack2(r, grp):
        v = efcb_v[r, pl.ds(grp * L, L)]
        lo = lax.bitcast_convert_type(lax.shift_left(v, 16), jnp.float32)
        hi = lax.bitcast_convert_type(v & jnp.int32(-65536), jnp.float32)
        return lo, hi

    def gather_descs(j, slot):
        da = pltpu.make_async_copy(a_hbm.at[idx_v.at[j]], a_v.at[slot],
                                   sem_a[slot])
        db = pltpu.make_async_copy(b_hbm.at[idx_v.at[IB + j]], b_v.at[slot],
                                   sem_b[slot])
        return da, db

    def efcb_desc(j, bidx):
        row0 = ebase + bidx * IB * K + j * K
        return pltpu.make_async_copy(efcb_hbm.at[pl.ds(row0, K)], efcb_v,
                                     sem_e)

    def store_descs(j, bidx, slot):
        row0 = ebase + bidx * IB * K + j * K
        dw = pltpu.make_async_copy(out_v.at[slot],
                                   efnew_hbm.at[pl.ds(row0, K)], sem_w[slot])
        dsc = pltpu.make_async_copy(out_v.at[slot],
                                    agg_sp.at[idx_v.at[IB + j]], sem_s[slot])
        return dw, dsc

    def fire(descs):
        for d in descs:
            d.start()

    def fire_stores(j, bidx, slot):
        dw, dsc = store_descs(j, bidx, slot)
        dw.start()
        dsc.start(add=True)

    def wait(descs):
        for d in descs:
            d.wait()

    def compute(slot):
        def crow(r, carry):
            for grp in range(4):
                a_lo = a_v[slot, r, pl.ds(32 * grp, L)]
                a_hi = a_v[slot, r, pl.ds(32 * grp + L, L)]
                b_lo = b_v[slot, r, pl.ds(32 * grp, L)]
                b_hi = b_v[slot, r, pl.ds(32 * grp + L, L)]
                ef_lo, ef_hi = unpack2(r, grp)
                cb_lo, cb_hi = unpack2(r, grp + 4)
                zero = jnp.zeros((L,), jnp.float32)
                out_v[slot, r, pl.ds(32 * grp, L)] = ef_lo + jnp.maximum(
                    a_lo + b_lo + cb_lo, zero)
                out_v[slot, r, pl.ds(32 * grp + L, L)] = ef_hi + jnp.maximum(
                    a_hi + b_hi + cb_hi, zero)
            return carry

        lax.fori_loop(0, K, crow, 0)

    def batch(bidx, carry):
        pltpu.sync_copy(eidx_hbm.at[w, bidx], idx_v)
        fire(gather_descs(0, 0))
        fire((efcb_desc(0, bidx),))

        def pair(h, carry2):
            j0 = 2 * h
            j1 = 2 * h + 1
            wait(gather_descs(j0, 0))
            fire(gather_descs(j1, 1))
            wait((efcb_desc(j0, bidx),))

            @pl.when(h >= 1)
            def _():
                wait(store_descs(j0 - 2, bidx, 0))

            compute(0)
            fire((efcb_desc(j1, bidx),))
            fire_stores(j0, bidx, 0)
            wait(gather_descs(j1, 1))

            @pl.when(h < HPB - 1)
            def _():
                fire(gather_descs(j1 + 1, 0))

            wait((efcb_desc(j1, bidx),))

            @pl.when(h >= 1)
            def _():
                wait(store_descs(j1 - 2, bidx, 1))

            compute(1)

            @pl.when(h < HPB - 1)
            def _():
                fire((efcb_desc(j1 + 1, bidx),))

            fire_stores(j1, bidx, 1)
            return carry2

        lax.fori_loop(0, HPB, pair, 0)
        wait(store_descs(IB - 2, bidx, 0))
        wait(store_descs(IB - 1, bidx, 1))
        return carry

    lax.fori_loop(0, NB, batch, 0)

    plsc.subcore_barrier()

    def ocopy(k, carry):
        r0 = base_row + k * ZR
        pltpu.sync_copy(agg_sp.at[pl.ds(r0, ZR)], agg_hbm.at[c, pl.ds(r0, ZR)])
        return carry

    lax.fori_loop(0, RPT // ZR, ocopy, 0)


@functools.cache
def _sc_call():
    return functools.partial(
        pl.kernel,
        out_type=(
            jax.ShapeDtypeStruct((E, D), jnp.float32),
            jax.ShapeDtypeStruct((NC, AGG_ROWS, D), jnp.float32),
        ),
        mesh=plsc.VectorSubcoreMesh(core_axis_name="c", subcore_axis_name="s",
                                    num_cores=NC, num_subcores=NS),
        scratch_types=[
            pltpu.VMEM((2 * IB, K), jnp.int32),
            pltpu.VMEM((2, K, D), jnp.float32),
            pltpu.VMEM((2, K, D), jnp.float32),
            pltpu.VMEM((K, D), jnp.int32),
            pltpu.VMEM((2, K, D), jnp.float32),
            pltpu.VMEM_SHARED((AGG_ROWS, D), jnp.float32),
            pltpu.SemaphoreType.DMA,
            pltpu.SemaphoreType.DMA,
            pltpu.SemaphoreType.DMA,
            pltpu.SemaphoreType.DMA,
            pltpu.SemaphoreType.DMA,
            pltpu.SemaphoreType.DMA,
            pltpu.SemaphoreType.DMA,
            pltpu.SemaphoreType.DMA,
            pltpu.SemaphoreType.DMA,
        ],
    )(_sc_body)


def kernel(x, edge_attr, edge_index, W_node_enc, b_node_enc, W_edge_enc,
           b_edge_enc, W_edge, b_edge, W_node, b_node):
    src = edge_index[0].reshape(NW, NB, IB, K)
    dst = edge_index[1].reshape(NW, NB, IB, K)
    eidx = jnp.concatenate([src, dst], axis=2)
    ce = jnp.array(_COLS_E, dtype=jnp.int32)
    co = jnp.array(_COLS_O, dtype=jnp.int32)
    W1 = W_edge[:D]
    W2 = W_edge[D:2 * D]
    W3 = W_edge[2 * D:]

    nf, A, B = _t1(x, W_node_enc, b_node_enc.reshape(1, D), W1, W2)
    efcb = _t2(edge_attr, W_edge_enc, b_edge_enc.reshape(1, D),
               W_edge_enc[:, ce], W_edge_enc[:, co],
               b_edge_enc[ce].reshape(1, DH), b_edge_enc[co].reshape(1, DH),
               W3[:, ce], W3[:, co],
               b_edge[ce].reshape(1, DH), b_edge[co].reshape(1, DH))
    ef_new, agg2 = _sc_call()(eidx, A, B, efcb)
    nf_new = _t3(nf, agg2[0, :N], agg2[1, :N], W_node[:D], W_node[D:],
                 b_node.reshape(1, D))
    return nf_new, ef_new


# speedup vs baseline: 3.4256x; 1.0369x over previous
import functools

import jax
import jax.numpy as jnp
from jax import lax
from jax.experimental import pallas as pl
from jax.experimental.pallas import tpu as pltpu
from jax.experimental.pallas import tpu_sc as plsc

N = 10000
E = 320000
ND = 128
ED = 16
D = 128
DH = D // 2

NC = 2
NS = 16
L = 16
NW = NC * NS
EPW = E // NW
K = 40
NCHUNK = EPW // K
IB = 50
NB = NCHUNK // IB
HPB = IB // 2
AGG_ROWS = 10112
RPT = AGG_ROWS // NS
ZR = 8

_COLS_E = [32 * j + m for j in range(4) for m in range(16)]
_COLS_O = [c + 16 for c in _COLS_E]


def _pack_pair(ve, vo):
    he = lax.bitcast_convert_type(ve.astype(jnp.bfloat16), jnp.uint16)
    ho = lax.bitcast_convert_type(vo.astype(jnp.bfloat16), jnp.uint16)
    packed = he.astype(jnp.uint32) | (ho.astype(jnp.uint32) << 16)
    return lax.bitcast_convert_type(packed, jnp.int32)


def _t12_body(x_ref, wne_ref, bne_ref, w1_ref, w2_ref,
              ea_ref, we_ref, be_ref, wee_ref, weo_ref, bee_ref, beo_ref,
              w3e_ref, w3o_ref, b3e_ref, b3o_ref,
              nf_ref, a_ref, b_ref, efcb_ref):
    i = pl.program_id(0)

    @pl.when(i == 0)
    def _():
        nf = jnp.dot(x_ref[...], wne_ref[...],
                     preferred_element_type=jnp.float32)
        nf = nf + bne_ref[...]
        nf_ref[...] = nf
        a_ref[...] = jnp.dot(nf, w1_ref[...],
                             preferred_element_type=jnp.float32)
        b_ref[...] = jnp.dot(nf, w2_ref[...],
                             preferred_element_type=jnp.float32)

    ea = ea_ref[...]
    ef = jnp.dot(ea, we_ref[...], preferred_element_type=jnp.float32)
    ef = ef + be_ref[...]
    ef_e = jnp.dot(ea, wee_ref[...], preferred_element_type=jnp.float32)
    ef_o = jnp.dot(ea, weo_ref[...], preferred_element_type=jnp.float32)
    efcb_ref[:, :DH] = _pack_pair(ef_e + bee_ref[...], ef_o + beo_ref[...])
    cb_e = jnp.dot(ef, w3e_ref[...], preferred_element_type=jnp.float32)
    cb_o = jnp.dot(ef, w3o_ref[...], preferred_element_type=jnp.float32)
    efcb_ref[:, DH:] = _pack_pair(cb_e + b3e_ref[...], cb_o + b3o_ref[...])


def _t3_body(nf_ref, agg_ref, wn1_ref, wn2_ref, bn_ref, out_ref):
    nf = nf_ref[...]
    agg = agg_ref[0] + agg_ref[1]
    h = (
        jnp.dot(nf, wn1_ref[...], preferred_element_type=jnp.float32)
        + jnp.dot(agg, wn2_ref[...], preferred_element_type=jnp.float32)
        + bn_ref[...]
    )
    out_ref[...] = nf + jnp.maximum(h, 0.0)


def _full(shape):
    return pl.BlockSpec(shape, lambda i: (0, 0))


def _t12(x, Wne, bne, W1, W2, ea, We, be, Wee, Weo, bee, beo,
         W3e, W3o, b3e, b3o):
    blk = 8000
    return pl.pallas_call(
        _t12_body,
        grid=(E // blk,),
        in_specs=[_full((N, ND)), _full((ND, D)), _full((1, D)),
                  _full((D, D)), _full((D, D)),
                  pl.BlockSpec((blk, ED), lambda i: (i, 0)),
                  _full((ED, D)), _full((1, D)),
                  _full((ED, DH)), _full((ED, DH)),
                  _full((1, DH)), _full((1, DH)),
                  _full((D, DH)), _full((D, DH)),
                  _full((1, DH)), _full((1, DH))],
        out_specs=[_full((N, D)), _full((N, D)), _full((N, D)),
                   pl.BlockSpec((blk, D), lambda i: (i, 0))],
        out_shape=[jax.ShapeDtypeStruct((N, D), jnp.float32),
                   jax.ShapeDtypeStruct((N, D), jnp.float32),
                   jax.ShapeDtypeStruct((N, D), jnp.float32),
                   jax.ShapeDtypeStruct((E, D), jnp.int32)],
    )(x, Wne, bne, W1, W2, ea, We, be, Wee, Weo, bee, beo, W3e, W3o, b3e, b3o)


def _t3(nf, agg2, Wn1, Wn2, bn):
    blk = 2000
    row = lambda d: pl.BlockSpec((blk, d), lambda i: (i, 0))
    return pl.pallas_call(
        _t3_body,
        grid=(N // blk,),
        in_specs=[row(D),
                  pl.BlockSpec((2, blk, D), lambda i: (0, i, 0)),
                  _full((D, D)), _full((D, D)), _full((1, D))],
        out_specs=row(D),
        out_shape=jax.ShapeDtypeStruct((N, D), jnp.float32),
    )(nf, agg2, Wn1, Wn2, bn)


def _sc_body(src_hbm, dst_hbm, a_hbm, b_hbm, efcb_hbm,
             efnew_hbm, agg_hbm,
             idx_v, a_v, b_v, efcb_v, out_v, agg_sp,
             sem_a0, sem_a1, sem_b0, sem_b1, sem_e, sem_w0, sem_w1,
             sem_s0, sem_s1):
    c = lax.axis_index("c")
    s = lax.axis_index("s")
    w = s * NC + c
    sem_a = (sem_a0, sem_a1)
    sem_b = (sem_b0, sem_b1)
    sem_w = (sem_w0, sem_w1)
    sem_s = (sem_s0, sem_s1)

    def zfill(i, carry):
        for j in range(D // L):
            out_v[0, i, pl.ds(j * L, L)] = jnp.zeros((L,), jnp.float32)
        return carry

    lax.fori_loop(0, ZR, zfill, 0)

    base_row = s * RPT

    def zcopy(k, carry):
        pltpu.sync_copy(out_v.at[0, pl.ds(0, ZR)],
                        agg_sp.at[pl.ds(base_row + k * ZR, ZR)])
        return carry

    lax.fori_loop(0, RPT // ZR, zcopy, 0)
    plsc.subcore_barrier()

    ebase = w * EPW

    def unpack2(r, grp):
        v = efcb_v[r, pl.ds(grp * L, L)]
        lo = lax.bitcast_convert_type(lax.shift_left(v, 16), jnp.float32)
        hi = lax.bitcast_convert_type(v & jnp.int32(-65536), jnp.float32)
        return lo, hi

    def gather_descs(j, slot):
        da = pltpu.make_async_copy(a_hbm.at[idx_v.at[j]], a_v.at[slot],
                                   sem_a[slot])
        db = pltpu.make_async_copy(b_hbm.at[idx_v.at[IB + j]], b_v.at[slot],
                                   sem_b[slot])
        return da, db

    def efcb_desc(j, bidx):
        row0 = ebase + bidx * IB * K + j * K
        return pltpu.make_async_copy(efcb_hbm.at[pl.ds(row0, K)], efcb_v,
                                     sem_e)

    def store_descs(j, bidx, slot):
        row0 = ebase + bidx * IB * K + j * K
        dw = pltpu.make_async_copy(out_v.at[slot],
                                   efnew_hbm.at[pl.ds(row0, K)], sem_w[slot])
        dsc = pltpu.make_async_copy(out_v.at[slot],
                                    agg_sp.at[idx_v.at[IB + j]], sem_s[slot])
        return dw, dsc

    def fire(descs):
        for d in descs:
            d.start()

    def fire_stores(j, bidx, slot):
        dw, dsc = store_descs(j, bidx, slot)
        dw.start()
        dsc.start(add=True)

    def wait(descs):
        for d in descs:
            d.wait()

    def compute(slot):
        def crow(r, carry):
            for grp in range(4):
                a_lo = a_v[slot, r, pl.ds(32 * grp, L)]
                a_hi = a_v[slot, r, pl.ds(32 * grp + L, L)]
                b_lo = b_v[slot, r, pl.ds(32 * grp, L)]
                b_hi = b_v[slot, r, pl.ds(32 * grp + L, L)]
                ef_lo, ef_hi = unpack2(r, grp)
                cb_lo, cb_hi = unpack2(r, grp + 4)
                zero = jnp.zeros((L,), jnp.float32)
                out_v[slot, r, pl.ds(32 * grp, L)] = ef_lo + jnp.maximum(
                    a_lo + b_lo + cb_lo, zero)
                out_v[slot, r, pl.ds(32 * grp + L, L)] = ef_hi + jnp.maximum(
                    a_hi + b_hi + cb_hi, zero)
            return carry

        lax.fori_loop(0, K, crow, 0)

    def batch(bidx, carry):
        pltpu.sync_copy(src_hbm.at[w, bidx], idx_v.at[pl.ds(0, IB)])
        pltpu.sync_copy(dst_hbm.at[w, bidx], idx_v.at[pl.ds(IB, IB)])
        fire(gather_descs(0, 0))
        fire((efcb_desc(0, bidx),))

        def pair(h, carry2):
            j0 = 2 * h
            j1 = 2 * h + 1
            wait(gather_descs(j0, 0))
            fire(gather_descs(j1, 1))
            wait((efcb_desc(j0, bidx),))

            @pl.when(h >= 1)
            def _():
                wait(store_descs(j0 - 2, bidx, 0))

            compute(0)
            fire((efcb_desc(j1, bidx),))
            fire_stores(j0, bidx, 0)
            wait(gather_descs(j1, 1))

            @pl.when(h < HPB - 1)
            def _():
                fire(gather_descs(j1 + 1, 0))

            wait((efcb_desc(j1, bidx),))

            @pl.when(h >= 1)
            def _():
                wait(store_descs(j1 - 2, bidx, 1))

            compute(1)

            @pl.when(h < HPB - 1)
            def _():
                fire((efcb_desc(j1 + 1, bidx),))

            fire_stores(j1, bidx, 1)
            return carry2

        lax.fori_loop(0, HPB, pair, 0)
        wait(store_descs(IB - 2, bidx, 0))
        wait(store_descs(IB - 1, bidx, 1))
        return carry

    lax.fori_loop(0, NB, batch, 0)

    plsc.subcore_barrier()

    def ocopy(k, carry):
        r0 = base_row + k * ZR
        pltpu.sync_copy(agg_sp.at[pl.ds(r0, ZR)], agg_hbm.at[c, pl.ds(r0, ZR)])
        return carry

    lax.fori_loop(0, RPT // ZR, ocopy, 0)


@functools.cache
def _sc_call():
    return functools.partial(
        pl.kernel,
        out_type=(
            jax.ShapeDtypeStruct((E, D), jnp.float32),
            jax.ShapeDtypeStruct((NC, AGG_ROWS, D), jnp.float32),
        ),
        mesh=plsc.VectorSubcoreMesh(core_axis_name="c", subcore_axis_name="s",
                                    num_cores=NC, num_subcores=NS),
        scratch_types=[
            pltpu.VMEM((2 * IB, K), jnp.int32),
            pltpu.VMEM((2, K, D), jnp.float32),
            pltpu.VMEM((2, K, D), jnp.float32),
            pltpu.VMEM((K, D), jnp.int32),
            pltpu.VMEM((2, K, D), jnp.float32),
            pltpu.VMEM_SHARED((AGG_ROWS, D), jnp.float32),
            pltpu.SemaphoreType.DMA,
            pltpu.SemaphoreType.DMA,
            pltpu.SemaphoreType.DMA,
            pltpu.SemaphoreType.DMA,
            pltpu.SemaphoreType.DMA,
            pltpu.SemaphoreType.DMA,
            pltpu.SemaphoreType.DMA,
            pltpu.SemaphoreType.DMA,
            pltpu.SemaphoreType.DMA,
        ],
    )(_sc_body)


def kernel(x, edge_attr, edge_index, W_node_enc, b_node_enc, W_edge_enc,
           b_edge_enc, W_edge, b_edge, W_node, b_node):
    src = edge_index[0].reshape(NW, NB, IB, K)
    dst = edge_index[1].reshape(NW, NB, IB, K)
    ce = jnp.array(_COLS_E, dtype=jnp.int32)
    co = jnp.array(_COLS_O, dtype=jnp.int32)
    W1 = W_edge[:D]
    W2 = W_edge[D:2 * D]
    W3 = W_edge[2 * D:]

    nf, A, B, efcb = _t12(
        x, W_node_enc, b_node_enc.reshape(1, D), W1, W2,
        edge_attr, W_edge_enc, b_edge_enc.reshape(1, D),
        W_edge_enc[:, ce], W_edge_enc[:, co],
        b_edge_enc[ce].reshape(1, DH), b_edge_enc[co].reshape(1, DH),
        W3[:, ce], W3[:, co],
        b_edge[ce].reshape(1, DH), b_edge[co].reshape(1, DH))
    ef_new, agg2 = _sc_call()(src, dst, A, B, efcb)
    nf_new = _t3(nf, agg2, W_node[:D], W_node[D:], b_node.reshape(1, D))
    return nf_new, ef_new


# speedup vs baseline: 3.4834x; 1.0169x over previous
import functools

import jax
import jax.numpy as jnp
from jax import lax
from jax.experimental import pallas as pl
from jax.experimental.pallas import tpu as pltpu
from jax.experimental.pallas import tpu_sc as plsc

N = 10000
E = 320000
ND = 128
ED = 16
D = 128
DH = D // 2

NC = 2
NS = 16
L = 16
NW = NC * NS
EPW = E // NW
K = 40
NCHUNK = EPW // K
IB = 50
NB = NCHUNK // IB
HPB = IB // 2
AGG_ROWS = 10112
RPT = AGG_ROWS // NS
ZR = 8

_COLS_E = [32 * j + m for j in range(4) for m in range(16)]
_COLS_O = [c + 16 for c in _COLS_E]


def _pack_pair(ve, vo):
    he = lax.bitcast_convert_type(ve.astype(jnp.bfloat16), jnp.uint16)
    ho = lax.bitcast_convert_type(vo.astype(jnp.bfloat16), jnp.uint16)
    packed = he.astype(jnp.uint32) | (ho.astype(jnp.uint32) << 16)
    return lax.bitcast_convert_type(packed, jnp.int32)


def _t12_body(x_ref, wne_ref, bne_ref, w1_ref, w2_ref,
              ea_ref, wbig_ref, bbig_ref,
              nf_ref, a_ref, b_ref, efcb_ref):
    i = pl.program_id(0)

    @pl.when(i == 0)
    def _():
        nf = jnp.dot(x_ref[...], wne_ref[...],
                     preferred_element_type=jnp.float32)
        nf = nf + bne_ref[...]
        nf_ref[...] = nf
        a_ref[...] = jnp.dot(nf, w1_ref[...],
                             preferred_element_type=jnp.float32)
        b_ref[...] = jnp.dot(nf, w2_ref[...],
                             preferred_element_type=jnp.float32)

    h = jnp.dot(ea_ref[...], wbig_ref[...],
                preferred_element_type=jnp.float32) + bbig_ref[...]
    efcb_ref[:, :DH] = _pack_pair(h[:, :DH], h[:, DH:2 * DH])
    efcb_ref[:, DH:] = _pack_pair(h[:, 2 * DH:3 * DH], h[:, 3 * DH:])


def _t3_body(nf_ref, agg_ref, wn1_ref, wn2_ref, bn_ref, out_ref):
    nf = nf_ref[...]
    agg = agg_ref[0] + agg_ref[1]
    h = (
        jnp.dot(nf, wn1_ref[...], preferred_element_type=jnp.float32)
        + jnp.dot(agg, wn2_ref[...], preferred_element_type=jnp.float32)
        + bn_ref[...]
    )
    out_ref[...] = nf + jnp.maximum(h, 0.0)


def _full(shape):
    return pl.BlockSpec(shape, lambda i: (0, 0))


def _t12(x, Wne, bne, W1, W2, ea, Wbig, bbig):
    blk = 8000
    return pl.pallas_call(
        _t12_body,
        grid=(E // blk,),
        in_specs=[_full((N, ND)), _full((ND, D)), _full((1, D)),
                  _full((D, D)), _full((D, D)),
                  pl.BlockSpec((blk, ED), lambda i: (i, 0)),
                  _full((ED, 2 * D)), _full((1, 2 * D))],
        out_specs=[_full((N, D)), _full((N, D)), _full((N, D)),
                   pl.BlockSpec((blk, D), lambda i: (i, 0))],
        out_shape=[jax.ShapeDtypeStruct((N, D), jnp.float32),
                   jax.ShapeDtypeStruct((N, D), jnp.float32),
                   jax.ShapeDtypeStruct((N, D), jnp.float32),
                   jax.ShapeDtypeStruct((E, D), jnp.int32)],
    )(x, Wne, bne, W1, W2, ea, Wbig, bbig)


def _t3(nf, agg2, Wn1, Wn2, bn):
    blk = 2000
    row = lambda d: pl.BlockSpec((blk, d), lambda i: (i, 0))
    return pl.pallas_call(
        _t3_body,
        grid=(N // blk,),
        in_specs=[row(D),
                  pl.BlockSpec((2, blk, D), lambda i: (0, i, 0)),
                  _full((D, D)), _full((D, D)), _full((1, D))],
        out_specs=row(D),
        out_shape=jax.ShapeDtypeStruct((N, D), jnp.float32),
    )(nf, agg2, Wn1, Wn2, bn)


def _sc_body(src_hbm, dst_hbm, a_hbm, b_hbm, efcb_hbm,
             efnew_hbm, agg_hbm,
             idx_v, a_v, b_v, efcb_v, out_v, agg_sp,
             sem_a0, sem_a1, sem_b0, sem_b1, sem_e, sem_w0, sem_w1,
             sem_s0, sem_s1):
    c = lax.axis_index("c")
    s = lax.axis_index("s")
    w = s * NC + c
    sem_a = (sem_a0, sem_a1)
    sem_b = (sem_b0, sem_b1)
    sem_w = (sem_w0, sem_w1)
    sem_s = (sem_s0, sem_s1)

    def zfill(i, carry):
        for j in range(D // L):
            out_v[0, i, pl.ds(j * L, L)] = jnp.zeros((L,), jnp.float32)
        return carry

    lax.fori_loop(0, ZR, zfill, 0)

    base_row = s * RPT

    def zcopy(k, carry):
        pltpu.sync_copy(out_v.at[0, pl.ds(0, ZR)],
                        agg_sp.at[pl.ds(base_row + k * ZR, ZR)])
        return carry

    lax.fori_loop(0, RPT // ZR, zcopy, 0)
    plsc.subcore_barrier()

    ebase = w * EPW

    def unpack2(r, grp):
        v = efcb_v[r, pl.ds(grp * L, L)]
        lo = lax.bitcast_convert_type(lax.shift_left(v, 16), jnp.float32)
        hi = lax.bitcast_convert_type(v & jnp.int32(-65536), jnp.float32)
        return lo, hi

    def gather_descs(j, slot):
        da = pltpu.make_async_copy(a_hbm.at[idx_v.at[j]], a_v.at[slot],
                                   sem_a[slot])
        db = pltpu.make_async_copy(b_hbm.at[idx_v.at[IB + j]], b_v.at[slot],
                                   sem_b[slot])
        return da, db

    def efcb_desc(j, bidx):
        row0 = ebase + bidx * IB * K + j * K
        return pltpu.make_async_copy(efcb_hbm.at[pl.ds(row0, K)], efcb_v,
                                     sem_e)

    def store_descs(j, bidx, slot):
        row0 = ebase + bidx * IB * K + j * K
        dw = pltpu.make_async_copy(out_v.at[slot],
                                   efnew_hbm.at[pl.ds(row0, K)], sem_w[slot])
        dsc = pltpu.make_async_copy(out_v.at[slot],
                                    agg_sp.at[idx_v.at[IB + j]], sem_s[slot])
        return dw, dsc

    def fire(descs):
        for d in descs:
            d.start()

    def fire_stores(j, bidx, slot):
        dw, dsc = store_descs(j, bidx, slot)
        dw.start()
        dsc.start(add=True)

    def wait(descs):
        for d in descs:
            d.wait()

    def compute(slot):
        def crow(r, carry):
            for grp in range(4):
                a_lo = a_v[slot, r, pl.ds(32 * grp, L)]
                a_hi = a_v[slot, r, pl.ds(32 * grp + L, L)]
                b_lo = b_v[slot, r, pl.ds(32 * grp, L)]
                b_hi = b_v[slot, r, pl.ds(32 * grp + L, L)]
                ef_lo, ef_hi = unpack2(r, grp)
                cb_lo, cb_hi = unpack2(r, grp + 4)
                zero = jnp.zeros((L,), jnp.float32)
                out_v[slot, r, pl.ds(32 * grp, L)] = ef_lo + jnp.maximum(
                    a_lo + b_lo + cb_lo, zero)
                out_v[slot, r, pl.ds(32 * grp + L, L)] = ef_hi + jnp.maximum(
                    a_hi + b_hi + cb_hi, zero)
            return carry

        lax.fori_loop(0, K, crow, 0)

    def batch(bidx, carry):
        pltpu.sync_copy(src_hbm.at[w, bidx], idx_v.at[pl.ds(0, IB)])
        pltpu.sync_copy(dst_hbm.at[w, bidx], idx_v.at[pl.ds(IB, IB)])
        fire(gather_descs(0, 0))
        fire((efcb_desc(0, bidx),))

        def pair(h, carry2):
            j0 = 2 * h
            j1 = 2 * h + 1
            wait(gather_descs(j0, 0))
            fire(gather_descs(j1, 1))
            wait((efcb_desc(j0, bidx),))

            @pl.when(h >= 1)
            def _():
                wait(store_descs(j0 - 2, bidx, 0))

            compute(0)
            fire((efcb_desc(j1, bidx),))
            fire_stores(j0, bidx, 0)
            wait(gather_descs(j1, 1))

            @pl.when(h < HPB - 1)
            def _():
                fire(gather_descs(j1 + 1, 0))

            wait((efcb_desc(j1, bidx),))

            @pl.when(h >= 1)
            def _():
                wait(store_descs(j1 - 2, bidx, 1))

            compute(1)

            @pl.when(h < HPB - 1)
            def _():
                fire((efcb_desc(j1 + 1, bidx),))

            fire_stores(j1, bidx, 1)
            return carry2

        lax.fori_loop(0, HPB, pair, 0)
        wait(store_descs(IB - 2, bidx, 0))
        wait(store_descs(IB - 1, bidx, 1))
        return carry

    lax.fori_loop(0, NB, batch, 0)

    plsc.subcore_barrier()

    def ocopy(k, carry):
        r0 = base_row + k * ZR
        pltpu.sync_copy(agg_sp.at[pl.ds(r0, ZR)], agg_hbm.at[c, pl.ds(r0, ZR)])
        return carry

    lax.fori_loop(0, RPT // ZR, ocopy, 0)


@functools.cache
def _sc_call():
    return functools.partial(
        pl.kernel,
        out_type=(
            jax.ShapeDtypeStruct((E, D), jnp.float32),
            jax.ShapeDtypeStruct((NC, AGG_ROWS, D), jnp.float32),
        ),
        mesh=plsc.VectorSubcoreMesh(core_axis_name="c", subcore_axis_name="s",
                                    num_cores=NC, num_subcores=NS),
        scratch_types=[
            pltpu.VMEM((2 * IB, K), jnp.int32),
            pltpu.VMEM((2, K, D), jnp.float32),
            pltpu.VMEM((2, K, D), jnp.float32),
            pltpu.VMEM((K, D), jnp.int32),
            pltpu.VMEM((2, K, D), jnp.float32),
            pltpu.VMEM_SHARED((AGG_ROWS, D), jnp.float32),
            pltpu.SemaphoreType.DMA,
            pltpu.SemaphoreType.DMA,
            pltpu.SemaphoreType.DMA,
            pltpu.SemaphoreType.DMA,
            pltpu.SemaphoreType.DMA,
            pltpu.SemaphoreType.DMA,
            pltpu.SemaphoreType.DMA,
            pltpu.SemaphoreType.DMA,
            pltpu.SemaphoreType.DMA,
        ],
    )(_sc_body)


def kernel(x, edge_attr, edge_index, W_node_enc, b_node_enc, W_edge_enc,
           b_edge_enc, W_edge, b_edge, W_node, b_node):
    src = edge_index[0].reshape(NW, NB, IB, K)
    dst = edge_index[1].reshape(NW, NB, IB, K)
    ce = jnp.array(_COLS_E, dtype=jnp.int32)
    co = jnp.array(_COLS_O, dtype=jnp.int32)
    W1 = W_edge[:D]
    W2 = W_edge[D:2 * D]
    W3 = W_edge[2 * D:]

    WeW3 = W_edge_enc @ W3
    b3f = b_edge_enc @ W3 + b_edge
    Wbig = jnp.concatenate(
        [W_edge_enc[:, ce], W_edge_enc[:, co], WeW3[:, ce], WeW3[:, co]],
        axis=1)
    bbig = jnp.concatenate(
        [b_edge_enc[ce], b_edge_enc[co], b3f[ce], b3f[co]]).reshape(1, 2 * D)
    nf, A, B, efcb = _t12(
        x, W_node_enc, b_node_enc.reshape(1, D), W1, W2,
        edge_attr, Wbig, bbig)
    ef_new, agg2 = _sc_call()(src, dst, A, B, efcb)
    nf_new = _t3(nf, agg2, W_node[:D], W_node[D:], b_node.reshape(1, D))
    return nf_new, ef_new


# speedup vs baseline: 3.7075x; 1.0643x over previous
import functools

import jax
import jax.numpy as jnp
from jax import lax
from jax.experimental import pallas as pl
from jax.experimental.pallas import tpu as pltpu
from jax.experimental.pallas import tpu_sc as plsc

N = 10000
E = 320000
ND = 128
ED = 16
D = 128
DH = D // 2

NC = 2
NS = 16
L = 16
NW = NC * NS
EPW = E // NW
K = 40
NCHUNK = EPW // K
IB = 50
NB = NCHUNK // IB
HPB = IB // 2
AGG_ROWS = 10112
RPT = AGG_ROWS // NS
ZR = 8

_COLS_E = [32 * j + m for j in range(4) for m in range(16)]
_COLS_O = [c + 16 for c in _COLS_E]


def _pack_pair(ve, vo):
    he = lax.bitcast_convert_type(ve.astype(jnp.bfloat16), jnp.uint16)
    ho = lax.bitcast_convert_type(vo.astype(jnp.bfloat16), jnp.uint16)
    packed = he.astype(jnp.uint32) | (ho.astype(jnp.uint32) << 16)
    return lax.bitcast_convert_type(packed, jnp.int32)


def _t12_body(x_ref, wne_ref, bne_ref, w1_ref, w2_ref,
              ea_ref, wbig_ref, bbig_ref,
              nf_ref, a_ref, b_ref, efcb_ref):
    i = pl.program_id(0)

    @pl.when(i == 0)
    def _():
        nf = jnp.dot(x_ref[...], wne_ref[...],
                     preferred_element_type=jnp.float32)
        nf = nf + bne_ref[...]
        nf_ref[...] = nf
        a_ref[...] = jnp.dot(nf, w1_ref[...],
                             preferred_element_type=jnp.float32)
        b_ref[...] = jnp.dot(nf, w2_ref[...],
                             preferred_element_type=jnp.float32)

    h = jnp.dot(ea_ref[...], wbig_ref[...],
                preferred_element_type=jnp.float32) + bbig_ref[...]
    efcb_ref[:, :DH] = _pack_pair(h[:, :DH], h[:, DH:2 * DH])
    efcb_ref[:, DH:] = _pack_pair(h[:, 2 * DH:3 * DH], h[:, 3 * DH:])


def _t3_body(nf_ref, agg_ref, wn1_ref, wn2_ref, bn_ref, out_ref):
    nf = nf_ref[...]
    agg = agg_ref[0] + agg_ref[1]
    h = (
        jnp.dot(nf, wn1_ref[...], preferred_element_type=jnp.float32)
        + jnp.dot(agg, wn2_ref[...], preferred_element_type=jnp.float32)
        + bn_ref[...]
    )
    out_ref[...] = nf + jnp.maximum(h, 0.0)


def _full(shape):
    return pl.BlockSpec(shape, lambda i: (0, 0))


def _t12(x, Wne, bne, W1, W2, ea, Wbig, bbig):
    blk = 8000
    return pl.pallas_call(
        _t12_body,
        grid=(E // blk,),
        in_specs=[_full((N, ND)), _full((ND, D)), _full((1, D)),
                  _full((D, D)), _full((D, D)),
                  pl.BlockSpec((blk, ED), lambda i: (i, 0)),
                  _full((ED, 2 * D)), _full((1, 2 * D))],
        out_specs=[_full((N, D)), _full((N, D)), _full((N, D)),
                   pl.BlockSpec((blk, D), lambda i: (i, 0))],
        out_shape=[jax.ShapeDtypeStruct((N, D), jnp.float32),
                   jax.ShapeDtypeStruct((N, D), jnp.float32),
                   jax.ShapeDtypeStruct((N, D), jnp.float32),
                   jax.ShapeDtypeStruct((E, D), jnp.int32)],
    )(x, Wne, bne, W1, W2, ea, Wbig, bbig)


def _t3(nf, agg2, Wn1, Wn2, bn):
    blk = 2000
    row = lambda d: pl.BlockSpec((blk, d), lambda i: (i, 0))
    return pl.pallas_call(
        _t3_body,
        grid=(N // blk,),
        in_specs=[row(D),
                  pl.BlockSpec((2, blk, D), lambda i: (0, i, 0)),
                  _full((D, D)), _full((D, D)), _full((1, D))],
        out_specs=row(D),
        out_shape=jax.ShapeDtypeStruct((N, D), jnp.float32),
    )(nf, agg2, Wn1, Wn2, bn)


def _sc_body(src_hbm, dst_hbm, a_hbm, b_hbm, efcb_hbm,
             efnew_hbm, agg_hbm,
             idx_v, a_v, b_v, efcb_v, out_v, agg_sp,
             sem_a0, sem_a1, sem_b0, sem_b1, sem_e, sem_w0, sem_w1,
             sem_s0, sem_s1):
    c = lax.axis_index("c")
    s = lax.axis_index("s")
    w = s * NC + c
    sem_a = (sem_a0, sem_a1)
    sem_b = (sem_b0, sem_b1)
    sem_w = (sem_w0, sem_w1)
    sem_s = (sem_s0, sem_s1)

    def zfill(i, carry):
        for slot in range(2):
            for j in range(D // L):
                out_v[slot, i, pl.ds(j * L, L)] = jnp.zeros((L,), jnp.float32)
        return carry

    lax.fori_loop(0, K, zfill, 0)

    base_row = s * RPT
    nzfull = RPT // K
    ztail = RPT - nzfull * K

    def zdesc(k):
        return pltpu.make_async_copy(
            out_v.at[k % 2], agg_sp.at[pl.ds(base_row + k * K, K)], sem_e)

    zt = pltpu.make_async_copy(
        out_v.at[0, pl.ds(0, ztail)],
        agg_sp.at[pl.ds(base_row + nzfull * K, ztail)], sem_e)
    for k in range(nzfull):
        zdesc(k).start()
    zt.start()
    for k in range(nzfull):
        zdesc(k).wait()
    zt.wait()
    plsc.subcore_barrier()

    ebase = w * EPW

    def unpack2(r, grp):
        v = efcb_v[r, pl.ds(grp * L, L)]
        lo = lax.bitcast_convert_type(lax.shift_left(v, 16), jnp.float32)
        hi = lax.bitcast_convert_type(v & jnp.int32(-65536), jnp.float32)
        return lo, hi

    def gather_descs(j, slot):
        da = pltpu.make_async_copy(a_hbm.at[idx_v.at[j]], a_v.at[slot],
                                   sem_a[slot])
        db = pltpu.make_async_copy(b_hbm.at[idx_v.at[IB + j]], b_v.at[slot],
                                   sem_b[slot])
        return da, db

    def efcb_desc(j, bidx):
        row0 = ebase + bidx * IB * K + j * K
        return pltpu.make_async_copy(efcb_hbm.at[pl.ds(row0, K)], efcb_v,
                                     sem_e)

    def store_descs(j, bidx, slot):
        row0 = ebase + bidx * IB * K + j * K
        dw = pltpu.make_async_copy(out_v.at[slot],
                                   efnew_hbm.at[pl.ds(row0, K)], sem_w[slot])
        dsc = pltpu.make_async_copy(out_v.at[slot],
                                    agg_sp.at[idx_v.at[IB + j]], sem_s[slot])
        return dw, dsc

    def fire(descs):
        for d in descs:
            d.start()

    def fire_stores(j, bidx, slot):
        dw, dsc = store_descs(j, bidx, slot)
        dw.start()
        dsc.start(add=True)

    def wait(descs):
        for d in descs:
            d.wait()

    def compute(slot):
        def crow(r, carry):
            for grp in range(4):
                a_lo = a_v[slot, r, pl.ds(32 * grp, L)]
                a_hi = a_v[slot, r, pl.ds(32 * grp + L, L)]
                b_lo = b_v[slot, r, pl.ds(32 * grp, L)]
                b_hi = b_v[slot, r, pl.ds(32 * grp + L, L)]
                ef_lo, ef_hi = unpack2(r, grp)
                cb_lo, cb_hi = unpack2(r, grp + 4)
                zero = jnp.zeros((L,), jnp.float32)
                out_v[slot, r, pl.ds(32 * grp, L)] = ef_lo + jnp.maximum(
                    a_lo + b_lo + cb_lo, zero)
                out_v[slot, r, pl.ds(32 * grp + L, L)] = ef_hi + jnp.maximum(
                    a_hi + b_hi + cb_hi, zero)
            return carry

        lax.fori_loop(0, K, crow, 0)

    def batch(bidx, carry):
        pltpu.sync_copy(src_hbm.at[w, bidx], idx_v.at[pl.ds(0, IB)])
        pltpu.sync_copy(dst_hbm.at[w, bidx], idx_v.at[pl.ds(IB, IB)])
        fire(gather_descs(0, 0))
        fire((efcb_desc(0, bidx),))

        def pair(h, carry2):
            j0 = 2 * h
            j1 = 2 * h + 1
            wait(gather_descs(j0, 0))
            fire(gather_descs(j1, 1))
            wait((efcb_desc(j0, bidx),))

            @pl.when(h >= 1)
            def _():
                wait(store_descs(j0 - 2, bidx, 0))

            compute(0)
            fire((efcb_desc(j1, bidx),))
            fire_stores(j0, bidx, 0)
            wait(gather_descs(j1, 1))

            @pl.when(h < HPB - 1)
            def _():
                fire(gather_descs(j1 + 1, 0))

            wait((efcb_desc(j1, bidx),))

            @pl.when(h >= 1)
            def _():
                wait(store_descs(j1 - 2, bidx, 1))

            compute(1)

            @pl.when(h < HPB - 1)
            def _():
                fire((efcb_desc(j1 + 1, bidx),))

            fire_stores(j1, bidx, 1)
            return carry2

        lax.fori_loop(0, HPB, pair, 0)
        wait(store_descs(IB - 2, bidx, 0))
        wait(store_descs(IB - 1, bidx, 1))
        return carry

    lax.fori_loop(0, NB, batch, 0)

    plsc.subcore_barrier()

    pltpu.sync_copy(agg_sp.at[pl.ds(base_row, RPT)],
                    agg_hbm.at[c, pl.ds(base_row, RPT)])


@functools.cache
def _sc_call():
    return functools.partial(
        pl.kernel,
        out_type=(
            jax.ShapeDtypeStruct((E, D), jnp.float32),
            jax.ShapeDtypeStruct((NC, AGG_ROWS, D), jnp.float32),
        ),
        mesh=plsc.VectorSubcoreMesh(core_axis_name="c", subcore_axis_name="s",
                                    num_cores=NC, num_subcores=NS),
        scratch_types=[
            pltpu.VMEM((2 * IB, K), jnp.int32),
            pltpu.VMEM((2, K, D), jnp.float32),
            pltpu.VMEM((2, K, D), jnp.float32),
            pltpu.VMEM((K, D), jnp.int32),
            pltpu.VMEM((2, K, D), jnp.float32),
            pltpu.VMEM_SHARED((AGG_ROWS, D), jnp.float32),
            pltpu.SemaphoreType.DMA,
            pltpu.SemaphoreType.DMA,
            pltpu.SemaphoreType.DMA,
            pltpu.SemaphoreType.DMA,
            pltpu.SemaphoreType.DMA,
            pltpu.SemaphoreType.DMA,
            pltpu.SemaphoreType.DMA,
            pltpu.SemaphoreType.DMA,
            pltpu.SemaphoreType.DMA,
        ],
    )(_sc_body)


def kernel(x, edge_attr, edge_index, W_node_enc, b_node_enc, W_edge_enc,
           b_edge_enc, W_edge, b_edge, W_node, b_node):
    src = edge_index[0].reshape(NW, NB, IB, K)
    dst = edge_index[1].reshape(NW, NB, IB, K)
    ce = jnp.array(_COLS_E, dtype=jnp.int32)
    co = jnp.array(_COLS_O, dtype=jnp.int32)
    W1 = W_edge[:D]
    W2 = W_edge[D:2 * D]
    W3 = W_edge[2 * D:]

    WeW3 = W_edge_enc @ W3
    b3f = b_edge_enc @ W3 + b_edge
    Wbig = jnp.concatenate(
        [W_edge_enc[:, ce], W_edge_enc[:, co], WeW3[:, ce], WeW3[:, co]],
        axis=1)
    bbig = jnp.concatenate(
        [b_edge_enc[ce], b_edge_enc[co], b3f[ce], b3f[co]]).reshape(1, 2 * D)
    nf, A, B, efcb = _t12(
        x, W_node_enc, b_node_enc.reshape(1, D), W1, W2,
        edge_attr, Wbig, bbig)
    ef_new, agg2 = _sc_call()(src, dst, A, B, efcb)
    nf_new = _t3(nf, agg2, W_node[:D], W_node[D:], b_node.reshape(1, D))
    return nf_new, ef_new
